# baseline (speedup 1.0000x reference)
import functools

import jax
import jax.numpy as jnp
from jax.experimental import pallas as pl

_N_CODEBOOKS = 8
_K = 8192
_E = 32
_W = 4096


def _bf16_store_round(v):
    u = jax.lax.bitcast_convert_type(v, jnp.uint32)
    u = (u + jnp.uint32(0x7FFF)) & jnp.uint32(0xFFFF0000)
    return jax.lax.bitcast_convert_type(u, jnp.float32)


def _rowsum32_lanes(sq):
    a = sq[:, 0:8] + sq[:, 8:16]
    a = a + sq[:, 16:24]
    a = a + sq[:, 24:32]
    b = a[:, 0:4] + a[:, 4:8]
    c = b[:, 0:2] + b[:, 2:4]
    return c[:, 0:1] + c[:, 1:2]


def _rowsum32_sublanes(sq):
    a = sq[0:8, :] + sq[8:16, :]
    a = a + sq[16:24, :]
    a = a + sq[24:32, :]
    b = a[0:4, :] + a[4:8, :]
    c = b[0:2, :] + b[2:4, :]
    return c[0:1, :] + c[1:2, :]


def _rvq_body(emb_ref, cb_ref, cbt_ref, out_ref, *, tn):
    r = emb_ref[...]
    quant = jnp.zeros_like(r)
    iota = jax.lax.broadcasted_iota(jnp.int32, (tn, _K), 1)
    iota_w = jax.lax.broadcasted_iota(jnp.int32, (tn, _W), 1)
    for i in range(_N_CODEBOOKS):
        cb = cb_ref[i]
        cbt = cbt_ref[i]
        n1 = _rowsum32_lanes(r * r)
        n2 = _rowsum32_sublanes(cbt * cbt)
        m = jax.lax.dot_general(
            r.astype(jnp.bfloat16), cb.astype(jnp.bfloat16),
            (((1,), (1,)), ((), ())),
            preferred_element_type=jnp.float32)
        d2 = n1 - 2.0 * m + n2
        run_v = jnp.full((tn, 1), jnp.inf, jnp.float32)
        run_i = jnp.zeros((tn, 1), jnp.int32)
        for c in range(_K // _W):
            ch = d2[:, c * _W:(c + 1) * _W]
            cmin = jnp.min(ch, axis=1, keepdims=True)
            cidx = jnp.min(
                jnp.where(ch == cmin, iota_w + c * _W, _K),
                axis=1, keepdims=True)
            upd = cmin < run_v
            run_i = jnp.where(upd, cidx, run_i)
            run_v = _bf16_store_round(jnp.where(upd, cmin, run_v))
        onehot = (iota == run_i).astype(jnp.float32)
        q = jax.lax.dot_general(
            onehot, cb, (((1,), (0,)), ((), ())),
            precision=jax.lax.Precision.HIGHEST,
            preferred_element_type=jnp.float32)
        quant = quant + q
        r = r - q
    out_ref[...] = quant


@jax.jit
def kernel(embeddings, codebooks):
    B, E, H, W = embeddings.shape
    flat = jnp.transpose(embeddings, (0, 2, 3, 1)).reshape(-1, E)
    cbt = jnp.transpose(codebooks, (0, 2, 1))
    n = flat.shape[0]
    tn = 256
    out = pl.pallas_call(
        functools.partial(_rvq_body, tn=tn),
        grid=(n // tn,),
        in_specs=[
            pl.BlockSpec((tn, E), lambda i: (i, 0)),
            pl.BlockSpec((_N_CODEBOOKS, _K, E), lambda i: (0, 0, 0)),
            pl.BlockSpec((_N_CODEBOOKS, _E, _K), lambda i: (0, 0, 0)),
        ],
        out_specs=pl.BlockSpec((tn, E), lambda i: (i, 0)),
        out_shape=jax.ShapeDtypeStruct((n, E), jnp.float32),
    )(flat, codebooks, cbt)
    out = out.reshape(B, H, W, E)
    return jnp.transpose(out, (0, 3, 1, 2))


# speedup vs baseline: 1.7255x; 1.7255x over previous
import functools

import jax
import jax.numpy as jnp
from jax.experimental import pallas as pl

_N_CODEBOOKS = 8
_K = 8192
_E = 32
_W = 4096
_CW = 512
_GW = 2048


def _bf16_store_round(v):
    u = jax.lax.bitcast_convert_type(v, jnp.uint32)
    u = (u + jnp.uint32(0x7FFF)) & jnp.uint32(0xFFFF0000)
    return jax.lax.bitcast_convert_type(u, jnp.float32)


def _rowsum32_lanes(sq):
    a = sq[:, 0:8] + sq[:, 8:16]
    a = a + sq[:, 16:24]
    a = a + sq[:, 24:32]
    b = a[:, 0:4] + a[:, 4:8]
    c = b[:, 0:2] + b[:, 2:4]
    return c[:, 0:1] + c[:, 1:2]


def _rowsum32_sublanes(sq):
    a = sq[0:8, :] + sq[8:16, :]
    a = a + sq[16:24, :]
    a = a + sq[24:32, :]
    b = a[0:4, :] + a[4:8, :]
    c = b[0:2, :] + b[2:4, :]
    return c[0:1, :] + c[1:2, :]


def _dot_nt(a, b):
    return jax.lax.dot_general(
        a, b, (((1,), (1,)), ((), ())), preferred_element_type=jnp.float32)


def _dot_nn(a, b):
    return jax.lax.dot_general(
        a, b, (((1,), (0,)), ((), ())), preferred_element_type=jnp.float32)


def _rvq_body(emb_ref, cb_ref, cbt_ref, out_ref, *, tn):
    r = emb_ref[...]
    quant = jnp.zeros_like(r)
    iota_cw = jax.lax.broadcasted_iota(jnp.int32, (tn, _CW), 1)
    iota_gw = jax.lax.broadcasted_iota(jnp.int32, (tn, _GW), 1)
    for i in range(_N_CODEBOOKS):
        cb = cb_ref[i]
        cbt = cbt_ref[i]
        hi16 = cb.astype(jnp.bfloat16)
        rem1 = cb - hi16.astype(jnp.float32)
        mid16 = rem1.astype(jnp.bfloat16)
        lo16 = (rem1 - mid16.astype(jnp.float32)).astype(jnp.bfloat16)
        r16 = r.astype(jnp.bfloat16)
        n1 = _rowsum32_lanes(r * r)
        n2 = _rowsum32_sublanes(cbt * cbt)
        run_v = None
        run_i = None
        for w in range(_K // _W):
            rm = jnp.full((tn, _CW), jnp.inf, jnp.float32)
            rk = jnp.full((tn, _CW), _K, jnp.int32)
            for s in range(_W // _CW):
                base = w * _W + s * _CW
                m_s = _dot_nt(r16, hi16[base:base + _CW])
                d2s = (n1 - 2.0 * m_s) + n2[:, base:base + _CW]
                lt = d2s < rm
                rk = jnp.where(lt, iota_cw + base, rk)
                rm = jnp.where(lt, d2s, rm)
            wmin = jnp.min(rm, axis=1, keepdims=True)
            widx = jnp.min(jnp.where(rm == wmin, rk, _K),
                           axis=1, keepdims=True)
            if w == 0:
                run_v, run_i = _bf16_store_round(wmin), widx
            else:
                upd = wmin < run_v
                run_i = jnp.where(upd, widx, run_i)
                run_v = _bf16_store_round(jnp.where(upd, wmin, run_v))
        q = jnp.zeros_like(r)
        for s in range(_K // _GW):
            base = s * _GW
            oh = (iota_gw + base == run_i).astype(jnp.bfloat16)
            q = q + _dot_nn(oh, hi16[base:base + _GW])
            q = q + _dot_nn(oh, mid16[base:base + _GW])
            q = q + _dot_nn(oh, lo16[base:base + _GW])
        quant = quant + q
        r = r - q
    out_ref[...] = quant


@jax.jit
def kernel(embeddings, codebooks):
    B, E, H, W = embeddings.shape
    flat = jnp.transpose(embeddings, (0, 2, 3, 1)).reshape(-1, E)
    cbt = jnp.transpose(codebooks, (0, 2, 1))
    n = flat.shape[0]
    tn = 512
    out = pl.pallas_call(
        functools.partial(_rvq_body, tn=tn),
        grid=(n // tn,),
        in_specs=[
            pl.BlockSpec((tn, E), lambda i: (i, 0)),
            pl.BlockSpec((_N_CODEBOOKS, _K, E), lambda i: (0, 0, 0)),
            pl.BlockSpec((_N_CODEBOOKS, _E, _K), lambda i: (0, 0, 0)),
        ],
        out_specs=pl.BlockSpec((tn, E), lambda i: (i, 0)),
        out_shape=jax.ShapeDtypeStruct((n, E), jnp.float32),
    )(flat, codebooks, cbt)
    out = out.reshape(B, H, W, E)
    return jnp.transpose(out, (0, 3, 1, 2))
